# baseline (speedup 1.0000x reference)
import functools

import jax
import jax.numpy as jnp
from jax import lax
from jax.experimental import pallas as pl
from jax.experimental.pallas import tpu as pltpu
from jax.experimental.pallas import tpu_sc as plsc

_NC, _NS, _L = 2, 16, 16
_NW = _NC * _NS
_CH = 16
_NBUF = 3
_DEPTH = 2


def _sc_gather_add(x_nc, idx_n, table):
    N, C = x_nc.shape
    n_per_w = N // _NW
    CH, NB, D = _CH, _NBUF, _DEPTH
    n_ch = n_per_w // CH
    mesh = plsc.VectorSubcoreMesh(core_axis_name="c", subcore_axis_name="s")

    scratch = [
        pltpu.VMEM((n_per_w,), jnp.int32),
        pltpu.VMEM((NB, CH, C), jnp.float32),
        pltpu.VMEM((NB, CH, C), jnp.float32),
        pltpu.SemaphoreType.DMA((NB,)),
        pltpu.SemaphoreType.DMA((NB,)),
        pltpu.SemaphoreType.DMA((NB,)),
    ]

    @functools.partial(
        pl.kernel,
        out_type=jax.ShapeDtypeStruct((N, C), jnp.float32),
        mesh=mesh,
        scratch_types=scratch,
    )
    def k(x_hbm, idx_hbm, tab_hbm, out_hbm, idx_v, xb, rb, sx, sr, so):
        wid = lax.axis_index("s") * _NC + lax.axis_index("c")
        base = wid * n_per_w
        pltpu.sync_copy(idx_hbm.at[pl.ds(base, n_per_w)], idx_v)

        def issue_x(ci, b):
            off = base + ci * CH
            pltpu.async_copy(x_hbm.at[pl.ds(off, CH)], xb.at[b], sx.at[b])

        def issue_g(ci, b):
            pltpu.async_copy(tab_hbm.at[idx_v.at[pl.ds(ci * CH, CH)]], rb.at[b],
                             sr.at[b])

        def wait_in(ci, b):
            off = base + ci * CH
            pltpu.make_async_copy(x_hbm.at[pl.ds(off, CH)], xb.at[b],
                                  sx.at[b]).wait()
            pltpu.make_async_copy(
                tab_hbm.at[idx_v.at[pl.ds(ci * CH, CH)]], rb.at[b],
                sr.at[b]).wait()

        def issue_out(ci, b):
            off = base + ci * CH
            pltpu.async_copy(xb.at[b], out_hbm.at[pl.ds(off, CH)], so.at[b])

        def wait_out(ci, b):
            off = base + ci * CH
            pltpu.make_async_copy(xb.at[b], out_hbm.at[pl.ds(off, CH)],
                                  so.at[b]).wait()

        def add_rows(b):
            def row(i, c2):
                for j in range(C // _L):
                    sl = pl.ds(j * _L, _L)
                    plsc.addupdate(xb.at[b, i, sl], rb[b, i, sl])
                return c2

            lax.fori_loop(0, CH, row, 0)

        issue_x(0, 0)
        for p in range(D):
            issue_g(p, p)

        def body(ci, carry):
            b = lax.rem(ci, NB)
            wait_in(ci, b)
            gci = ci + D
            gb = lax.rem(gci, NB)

            @pl.when(gci < n_ch)
            def _():
                issue_g(gci, gb)

            xci = ci + 1
            xbuf = lax.rem(xci, NB)

            @pl.when(xci < n_ch)
            def _():
                @pl.when(xci >= NB)
                def _():
                    wait_out(xci - NB, xbuf)

                issue_x(xci, xbuf)

            add_rows(b)
            issue_out(ci, b)
            return carry

        lax.fori_loop(0, n_ch, body, 0)
        for t in range(NB):
            ci = n_ch - NB + t
            wait_out(ci, ci % NB)

    return k(x_nc, idx_n, table)


def kernel(x_btc, times_bt, embeddings_tc, offset):
    B, T, C = x_btc.shape
    x = x_btc.reshape(B * T, C)
    idx = times_bt.reshape(B * T).astype(jnp.int32)
    out = _sc_gather_add(x, idx, embeddings_tc)
    return out.reshape(B, T, C)


# speedup vs baseline: 1.0492x; 1.0492x over previous
import functools

import jax
import jax.numpy as jnp
from jax import lax
from jax.experimental import pallas as pl
from jax.experimental.pallas import tpu as pltpu
from jax.experimental.pallas import tpu_sc as plsc

_NC, _NS, _L = 2, 16, 16
_NW = _NC * _NS
_CH = 16
_NBUF = 3
_DEPTH = 2


def _sc_gather_add(x_nc, idx_n, table):
    N, C = x_nc.shape
    n_per_w = N // _NW
    CH, NB, D = _CH, _NBUF, _DEPTH
    n_ch = n_per_w // CH
    mesh = plsc.VectorSubcoreMesh(core_axis_name="c", subcore_axis_name="s")

    scratch = [
        pltpu.VMEM((n_per_w,), jnp.int32),
        pltpu.VMEM((NB, CH, C), jnp.float32),
        pltpu.VMEM((NB, CH, C), jnp.float32),
        pltpu.SemaphoreType.DMA((NB,)),
        pltpu.SemaphoreType.DMA((NB,)),
        pltpu.SemaphoreType.DMA((NB,)),
    ]

    @functools.partial(
        pl.kernel,
        out_type=jax.ShapeDtypeStruct((N, C), jnp.float32),
        mesh=mesh,
        scratch_types=scratch,
    )
    def k(x_hbm, idx_hbm, tab_hbm, out_hbm, idx_v, xb, rb, sx, sr, so):
        wid = lax.axis_index("s") * _NC + lax.axis_index("c")
        base = wid * n_per_w
        pltpu.sync_copy(idx_hbm.at[pl.ds(base, n_per_w)], idx_v)

        def issue_in(ci, b):
            off = base + ci * CH
            pltpu.async_copy(x_hbm.at[pl.ds(off, CH)], xb.at[b], sx.at[b])
            pltpu.async_copy(tab_hbm.at[idx_v.at[pl.ds(ci * CH, CH)]], rb.at[b],
                             sr.at[b])

        def wait_in(ci, b):
            off = base + ci * CH
            pltpu.make_async_copy(x_hbm.at[pl.ds(off, CH)], xb.at[b],
                                  sx.at[b]).wait()
            pltpu.make_async_copy(
                tab_hbm.at[idx_v.at[pl.ds(ci * CH, CH)]], rb.at[b],
                sr.at[b]).wait()

        def issue_out(ci, b):
            off = base + ci * CH
            pltpu.async_copy(xb.at[b], out_hbm.at[pl.ds(off, CH)], so.at[b])

        def wait_out(ci, b):
            off = base + ci * CH
            pltpu.make_async_copy(xb.at[b], out_hbm.at[pl.ds(off, CH)],
                                  so.at[b]).wait()

        def add_rows(b):
            def row(i, c2):
                for j in range(C // _L):
                    sl = pl.ds(j * _L, _L)
                    plsc.addupdate(xb.at[b, i, sl], rb[b, i, sl])
                return c2

            lax.fori_loop(0, CH, row, 0)

        for p in range(D):
            issue_in(p, p)

        def body(ci, carry):
            b = lax.rem(ci, NB)
            wait_in(ci, b)
            nci = ci + D
            nb = lax.rem(nci, NB)

            @pl.when(nci < n_ch)
            def _():
                @pl.when(nci >= NB)
                def _():
                    wait_out(nci - NB, nb)

                issue_in(nci, nb)

            add_rows(b)
            issue_out(ci, b)
            return carry

        lax.fori_loop(0, n_ch, body, 0)
        for t in range(NB):
            ci = n_ch - NB + t
            wait_out(ci, ci % NB)

    return k(x_nc, idx_n, table)


def kernel(x_btc, times_bt, embeddings_tc, offset):
    B, T, C = x_btc.shape
    x = x_btc.reshape(B * T, C)
    idx = times_bt.reshape(B * T).astype(jnp.int32)
    out = _sc_gather_add(x, idx, embeddings_tc)
    return out.reshape(B, T, C)
